# baseline (speedup 1.0000x reference)
import jax
import jax.numpy as jnp
from jax.experimental import pallas as pl
from jax.experimental.pallas import tpu as pltpu

CLIP = 64
ROWS_PER_BLOCK = 8


def _build_band_kernel(table_ref, e_ref, *, S, C, D):
    e_ref[0 : S - C, :] = jnp.broadcast_to(table_ref[0:1, :], (S - C, D))
    e_ref[S - C : S - 1 + C, :] = table_ref[1 : 2 * C, :]
    e_ref[S - 1 + C :, :] = jnp.broadcast_to(table_ref[2 * C : 2 * C + 1, :], (S - C + 1, D))


def _window_kernel(e_ref, out_ref, *, S, BI):
    i = pl.program_id(0)
    for bi in range(BI):
        row = i * BI + bi
        out_ref[bi] = e_ref[pl.ds(S - 1 - row, S), :]


def _rel_pos_encoding(table, S, C, D, interpret=False):
    BI = ROWS_PER_BLOCK
    band = pl.pallas_call(
        lambda t, e: _build_band_kernel(t, e, S=S, C=C, D=D),
        in_specs=[pl.BlockSpec(memory_space=pltpu.VMEM)],
        out_specs=pl.BlockSpec(memory_space=pltpu.VMEM),
        out_shape=jax.ShapeDtypeStruct((2 * S, D), table.dtype),
        interpret=interpret,
    )(table)
    return pl.pallas_call(
        lambda e, o: _window_kernel(e, o, S=S, BI=BI),
        grid=(S // BI,),
        in_specs=[pl.BlockSpec((2 * S, D), lambda i: (0, 0))],
        out_specs=pl.BlockSpec((BI, S, D), lambda i: (i, 0, 0)),
        out_shape=jax.ShapeDtypeStruct((S, S, D), table.dtype),
        compiler_params=pltpu.CompilerParams(
            dimension_semantics=("parallel",),
        ),
        interpret=interpret,
    )(band)


def kernel(x, encoding_matrix):
    S = x.shape[1]
    D = encoding_matrix.shape[1]
    return _rel_pos_encoding(encoding_matrix, S, CLIP, D)
